# Initial kernel scaffold; baseline (speedup 1.0000x reference)
import jax
import jax.numpy as jnp
from jax.experimental import pallas as pl


def kernel(probabilities, bin_params):
    raise NotImplementedError("write your pallas kernel here")


# speedup vs baseline: 8.8442x; 8.8442x over previous
import functools

import jax
import jax.numpy as jnp
import numpy as np
from jax import lax
from jax.experimental import pallas as pl
from jax.experimental.pallas import tpu as pltpu
from jax.experimental.pallas import tpu_sc as plsc

NUM_BINS = 10
MIN_CLAMP = 0.01
MAX_CLAMP = 100.0
BATCH = 16384
NUM_CLASSES = 1000

_EDGE_BITS = np.array(
    [0x00000000, 0x3DCCCCCD, 0x3E4CCCCD, 0x3E99999A, 0x3ECCCCCD, 0x3F000000,
     0x3F19999A, 0x3F333333, 0x3F4CCCCD, 0x3F666667, 0x3F800000],
    dtype=np.uint32)
_EDGES = tuple(_EDGE_BITS.view(np.float32).tolist())

_NC, _NS = 2, 16
_NW = _NC * _NS
_LANES = 16
_ROWS_PER_W = BATCH // _NW
_GROUPS = _ROWS_PER_W // _LANES


def _tec_kernel(prob_hbm, params_hbm, out_hbm, buf, params_v, out_v):
    cid = lax.axis_index("c")
    sid = lax.axis_index("s")
    wid = cid * _NS + sid
    row_base = wid * _ROWS_PER_W

    pltpu.sync_copy(params_hbm, params_v)
    row_off = lax.iota(jnp.int32, _LANES) * NUM_CLASSES

    def group(g, _):
        pltpu.sync_copy(
            prob_hbm.at[pl.ds((row_base + g * _LANES) * NUM_CLASSES,
                              _LANES * NUM_CLASSES)],
            buf)

        def col(j, acc):
            v = plsc.load_gather(buf, [row_off + j])
            return jnp.maximum(acc, v)

        m = lax.fori_loop(0, NUM_CLASSES, col,
                          jnp.full((_LANES,), -jnp.inf, jnp.float32))

        cnt = jnp.zeros((_LANES,), jnp.int32)
        for e in _EDGES:
            cnt = cnt + jnp.where(m > jnp.float32(e),
                                  jnp.int32(1), jnp.int32(0))
        bin_idx = jnp.clip(cnt - 1, 0, NUM_BINS - 1)
        bv = plsc.load_gather(params_v, [bin_idx])
        temp = jnp.clip(bv * bv, MIN_CLAMP, MAX_CLAMP)
        out_v[pl.ds(g * _LANES, _LANES)] = m * (1.0 / temp)
        return 0

    lax.fori_loop(0, _GROUPS, group, 0)
    pltpu.sync_copy(out_v, out_hbm.at[pl.ds(row_base, _ROWS_PER_W)])


@functools.partial(jax.jit, static_argnames=())
def _run(prob_flat, params16):
    mesh = plsc.VectorSubcoreMesh(core_axis_name="c", subcore_axis_name="s",
                                  num_cores=_NC, num_subcores=_NS)
    f = pl.kernel(
        _tec_kernel,
        out_type=jax.ShapeDtypeStruct((BATCH,), jnp.float32),
        mesh=mesh,
        scratch_types=[
            pltpu.VMEM((_LANES * NUM_CLASSES,), jnp.float32),
            pltpu.VMEM((_LANES,), jnp.float32),
            pltpu.VMEM((_ROWS_PER_W,), jnp.float32),
        ],
        compiler_params=pltpu.CompilerParams(needs_layout_passes=False),
    )
    return f(prob_flat, params16)


def kernel(probabilities, bin_params):
    prob_flat = probabilities.reshape(-1)
    params16 = jnp.concatenate(
        [bin_params, jnp.zeros((_LANES - NUM_BINS,), jnp.float32)])
    output = _run(prob_flat, params16)
    calibrated = jnp.zeros((BATCH, NUM_CLASSES), jnp.float32)
    return (output, calibrated)


# speedup vs baseline: 14.7053x; 1.6627x over previous
import functools

import jax
import jax.numpy as jnp
import numpy as np
from jax import lax
from jax.experimental import pallas as pl
from jax.experimental.pallas import tpu as pltpu
from jax.experimental.pallas import tpu_sc as plsc

NUM_BINS = 10
MIN_CLAMP = 0.01
MAX_CLAMP = 100.0
BATCH = 16384
NUM_CLASSES = 1000

_EDGE_BITS = np.array(
    [0x00000000, 0x3DCCCCCD, 0x3E4CCCCD, 0x3E99999A, 0x3ECCCCCD, 0x3F000000,
     0x3F19999A, 0x3F333333, 0x3F4CCCCD, 0x3F666667, 0x3F800000],
    dtype=np.uint32)
_EDGES = tuple(_EDGE_BITS.view(np.float32).tolist())

_NC, _NS = 2, 16
_NW = _NC * _NS
_LANES = 16
_ROWS_PER_W = BATCH // _NW
_GROUPS = _ROWS_PER_W // _LANES


_UNROLL = 8
_STEPS = NUM_CLASSES // _UNROLL


def _tec_kernel(prob_hbm, params_hbm, out_hbm, buf0, buf1, params_v, out_v,
                sem0, sem1):
    cid = lax.axis_index("c")
    sid = lax.axis_index("s")
    wid = cid * _NS + sid
    row_base = wid * _ROWS_PER_W

    pltpu.sync_copy(params_hbm, params_v)
    row_off = lax.iota(jnp.int32, _LANES) * NUM_CLASSES

    bufs = (buf0, buf1)
    sems = (sem0, sem1)

    def start(g):
        return pltpu.async_copy(
            prob_hbm.at[pl.ds((row_base + g * _LANES) * NUM_CLASSES,
                              _LANES * NUM_CLASSES)],
            bufs[g % 2], sems[g % 2])

    pending = start(0)
    for g in range(_GROUPS):
        pending.wait()
        if g + 1 < _GROUPS:
            pending = start(g + 1)
        buf = bufs[g % 2]

        def col(i, accs, buf=buf):
            base = i * _UNROLL
            return tuple(
                jnp.maximum(a, plsc.load_gather(buf, [row_off + (base + k)]))
                for k, a in enumerate(accs))

        accs = lax.fori_loop(
            0, _STEPS, col,
            tuple(jnp.full((_LANES,), -jnp.inf, jnp.float32)
                  for _ in range(_UNROLL)))
        while len(accs) > 1:
            accs = tuple(jnp.maximum(accs[2 * t], accs[2 * t + 1])
                         for t in range(len(accs) // 2))
        m = accs[0]

        cnt = jnp.zeros((_LANES,), jnp.int32)
        for e in _EDGES:
            cnt = cnt + jnp.where(m > jnp.float32(e),
                                  jnp.int32(1), jnp.int32(0))
        bin_idx = jnp.clip(cnt - 1, 0, NUM_BINS - 1)
        bv = plsc.load_gather(params_v, [bin_idx])
        temp = jnp.clip(bv * bv, MIN_CLAMP, MAX_CLAMP)
        out_v[pl.ds(g * _LANES, _LANES)] = m * (1.0 / temp)

    pltpu.sync_copy(out_v, out_hbm.at[pl.ds(row_base, _ROWS_PER_W)])


@functools.partial(jax.jit, static_argnames=())
def _run(prob_flat, params16):
    mesh = plsc.VectorSubcoreMesh(core_axis_name="c", subcore_axis_name="s",
                                  num_cores=_NC, num_subcores=_NS)
    f = pl.kernel(
        _tec_kernel,
        out_type=jax.ShapeDtypeStruct((BATCH,), jnp.float32),
        mesh=mesh,
        scratch_types=[
            pltpu.VMEM((_LANES * NUM_CLASSES,), jnp.float32),
            pltpu.VMEM((_LANES * NUM_CLASSES,), jnp.float32),
            pltpu.VMEM((_LANES,), jnp.float32),
            pltpu.VMEM((_ROWS_PER_W,), jnp.float32),
            pltpu.SemaphoreType.DMA,
            pltpu.SemaphoreType.DMA,
        ],
        compiler_params=pltpu.CompilerParams(needs_layout_passes=False),
    )
    return f(prob_flat, params16)


def kernel(probabilities, bin_params):
    prob_flat = probabilities.reshape(-1)
    params16 = jnp.concatenate(
        [bin_params, jnp.zeros((_LANES - NUM_BINS,), jnp.float32)])
    output = _run(prob_flat, params16)
    calibrated = jnp.zeros((BATCH, NUM_CLASSES), jnp.float32)
    return (output, calibrated)
